# Initial kernel scaffold; baseline (speedup 1.0000x reference)
import jax
import jax.numpy as jnp
from jax.experimental import pallas as pl


def kernel(x_drug, x_protein, Wp_drug, bp_drug, Wp_protein, bp_protein, Wm_binds, Wr_binds, b_binds, Wm_rev, Wr_rev, b_rev, Wl_drug, bl_drug, Wl_protein, bl_protein, edge_index_binds, edge_index_rev):
    raise NotImplementedError("write your pallas kernel here")


# speedup vs baseline: 1.2605x; 1.2605x over previous
import functools

import jax
import jax.numpy as jnp
from jax import lax
from jax.experimental import pallas as pl
from jax.experimental.pallas import tpu as pltpu
from jax.experimental.pallas import tpu_sc as plsc

_N = 50000
_D = 128
_E = 300000

_NP = 51200
_EP = 311296
_TILE_E = _EP // 16
_NVEC = _TILE_E // 16

_DUMP = 1024
_NTOT = _NP + _DUMP
_NCHUNK = 6
_RCH = _NTOT // _NCHUNK
_ACC_ROWS = _RCH + 64
_ZR = 68
_SB = 2432
_SBV = _SB // 16
_CAP = _SB + 256

_RB = 1024

_mesh = plsc.VectorSubcoreMesh(
    core_axis_name="c", subcore_axis_name="s", num_cores=2, num_subcores=16)


def _conv_body(hd_hbm, hp_hbm, srcb_hbm, dstb_hbm, srcr_hbm, dstr_hbm,
               aggp_hbm, aggd_hbm,
               src_v, dst_v, csrc_v, cdst_v, crow_v, rows_v, z_v, acc_sh, sem):
  sid = lax.axis_index("s")
  ebase = sid * _TILE_E

  def zero_z(i, carry):
    for k in range(8):
      z_v[i, pl.ds(k * 16, 16)] = jnp.zeros((16,), jnp.float32)
    return carry

  lax.fori_loop(0, _ZR, zero_z, 0, unroll=2)

  def step(h_hbm, j):
    for k in range(8):
      crow_v[0, pl.ds(k * 16, 16)] = cdst_v[pl.ds(j * 128 + k * 16, 16)]
    gather = pltpu.make_async_copy(
        h_hbm.at[csrc_v.at[pl.ds(j * 128, 128)]], rows_v, sem)
    gather.start()
    gather.wait()
    pltpu.sync_copy(rows_v, acc_sh.at[crow_v.at[0]], add=True)

  def one_relation(h_hbm, src_hbm, dst_hbm, out_hbm):
    for c in range(_NCHUNK):
      lo = c * _RCH
      for q in range(8):
        pltpu.sync_copy(z_v, acc_sh.at[pl.ds(sid * (_RCH // 16) + q * _ZR, _ZR)])
      plsc.subcore_barrier()

      def sub_batch(sb, cur):
        pltpu.sync_copy(src_hbm.at[pl.ds(ebase + sb * _SB, _SB)], src_v)
        pltpu.sync_copy(dst_hbm.at[pl.ds(ebase + sb * _SB, _SB)], dst_v)

        def compact(i, cnt):
          s = src_v[pl.ds(i * 16, 16)]
          d = dst_v[pl.ds(i * 16, 16)]
          dl = d - lo
          m = (dl >= 0) & (dl < _RCH)
          plsc.store_compressed(csrc_v.at[pl.ds(cnt, 16)], s, mask=m)
          plsc.store_compressed(cdst_v.at[pl.ds(cnt, 16)], dl, mask=m)
          return cnt + jnp.sum(m.astype(jnp.int32))

        cur = lax.fori_loop(0, _SBV, compact, cur, unroll=2)
        nfull = cur // 128

        def flush(j, carry):
          step(h_hbm, j)
          return carry

        lax.fori_loop(0, nfull, flush, 0)
        base = nfull * 128
        for k in range(8):
          csrc_v[pl.ds(k * 16, 16)] = csrc_v[pl.ds(base + k * 16, 16)]
          cdst_v[pl.ds(k * 16, 16)] = cdst_v[pl.ds(base + k * 16, 16)]
        return cur - base

      cur = lax.fori_loop(0, _TILE_E // _SB, sub_batch, jnp.int32(0))

      dump = jnp.full((16,), _RCH, jnp.int32) + lax.iota(jnp.int32, 16)
      zero16 = jnp.zeros((16,), jnp.int32)
      for k in range(8):
        csrc_v[pl.ds(cur + k * 16, 16)] = zero16
        cdst_v[pl.ds(cur + k * 16, 16)] = dump

      def flush_tail(j, carry):
        step(h_hbm, j)
        return carry

      lax.fori_loop(0, (cur + 127) // 128, flush_tail, 0)
      plsc.subcore_barrier()
      nrows = (_RCH if c < _NCHUNK - 1 else _RCH - _DUMP) // 16
      pltpu.sync_copy(
          acc_sh.at[pl.ds(sid * nrows, nrows)],
          out_hbm.at[pl.ds(lo + sid * nrows, nrows)])
      plsc.subcore_barrier()

  cid = lax.axis_index("c")

  @pl.when(cid == 0)
  def _():
    one_relation(hd_hbm, srcb_hbm, dstb_hbm, aggp_hbm)

  @pl.when(cid == 1)
  def _():
    one_relation(hp_hbm, srcr_hbm, dstr_hbm, aggd_hbm)


def _conv_call(hd, hp, srcb, dstb, srcr, dstr):
  return pl.kernel(
      _conv_body,
      out_type=(jax.ShapeDtypeStruct((_NP, _D), jnp.float32),
                jax.ShapeDtypeStruct((_NP, _D), jnp.float32)),
      mesh=_mesh,
      scratch_types=[
          pltpu.VMEM((_SB,), jnp.int32),
          pltpu.VMEM((_SB,), jnp.int32),
          pltpu.VMEM((_CAP,), jnp.int32),
          pltpu.VMEM((_CAP,), jnp.int32),
          pltpu.VMEM((1, 128), jnp.int32),
          pltpu.VMEM((128, _D), jnp.float32),
          pltpu.VMEM((_ZR, _D), jnp.float32),
          pltpu.VMEM_SHARED((_ACC_ROWS, _D), jnp.float32),
          pltpu.SemaphoreType.DMA,
      ],
      compiler_params=pltpu.CompilerParams(needs_layout_passes=False),
  )(hd, hp, srcb, dstb, srcr, dstr)


def _proj_body(x_ref, w_ref, b_ref, o_ref, *, relu):
  y = jnp.dot(x_ref[...], w_ref[...], preferred_element_type=jnp.float32)
  y = y + b_ref[...]
  o_ref[...] = jnp.maximum(y, 0.0) if relu else y


def _proj(x, w, b, relu):
  n = x.shape[0]
  return pl.pallas_call(
      functools.partial(_proj_body, relu=relu),
      grid=(n // _RB,),
      in_specs=[
          pl.BlockSpec((_RB, _D), lambda i: (i, 0)),
          pl.BlockSpec((_D, _D), lambda i: (0, 0)),
          pl.BlockSpec((1, _D), lambda i: (0, 0)),
      ],
      out_specs=pl.BlockSpec((_RB, _D), lambda i: (i, 0)),
      out_shape=jax.ShapeDtypeStruct((n, _D), jnp.float32),
  )(x, w, b.reshape(1, _D))


def _combine_body(agg_ref, deg_ref, h_ref, wm_ref, wr_ref, b_ref, o_ref):
  inv = 1.0 / jnp.maximum(deg_ref[:, :1], 1.0)
  y = jnp.dot(agg_ref[...] * inv, wm_ref[...],
              preferred_element_type=jnp.float32)
  y = y + jnp.dot(h_ref[...], wr_ref[...], preferred_element_type=jnp.float32)
  o_ref[...] = jnp.maximum(y + b_ref[...], 0.0)


def _combine(agg, deg, h, wm, wr, b):
  return pl.pallas_call(
      _combine_body,
      grid=(_NP // _RB,),
      in_specs=[
          pl.BlockSpec((_RB, _D), lambda i: (i, 0)),
          pl.BlockSpec((_RB, _D), lambda i: (i, 0)),
          pl.BlockSpec((_RB, _D), lambda i: (i, 0)),
          pl.BlockSpec((_D, _D), lambda i: (0, 0)),
          pl.BlockSpec((_D, _D), lambda i: (0, 0)),
          pl.BlockSpec((1, _D), lambda i: (0, 0)),
      ],
      out_specs=pl.BlockSpec((_RB, _D), lambda i: (i, 0)),
      out_shape=jax.ShapeDtypeStruct((_NP, _D), jnp.float32),
  )(agg, deg, h, wm, wr, b.reshape(1, _D))


def _pad_edges(ei):
  npad = _EP - _E
  pad_src = jnp.zeros((npad,), jnp.int32)
  pad_dst = (_NP + (jnp.arange(npad, dtype=jnp.int32) % _DUMP)).astype(jnp.int32)
  src = jnp.concatenate([ei[0], pad_src])
  dst = jnp.concatenate([ei[1], pad_dst])
  return src, dst


def kernel(x_drug, x_protein, Wp_drug, bp_drug, Wp_protein, bp_protein,
           Wm_binds, Wr_binds, b_binds, Wm_rev, Wr_rev, b_rev,
           Wl_drug, bl_drug, Wl_protein, bl_protein,
           edge_index_binds, edge_index_rev):
  pad = _NP - _N
  xd = jnp.pad(x_drug, ((0, pad), (0, 0)))
  xp = jnp.pad(x_protein, ((0, pad), (0, 0)))
  srcb, dstb = _pad_edges(edge_index_binds)
  srcr, dstr = _pad_edges(edge_index_rev)

  ones = jnp.ones((_NP, _D), jnp.float32)
  degp, degd = _conv_call(ones, ones, srcb, dstb, srcr, dstr)

  hd = _proj(xd, Wp_drug, bp_drug, relu=True)
  hp = _proj(xp, Wp_protein, bp_protein, relu=True)

  for l in range(Wm_binds.shape[0]):
    aggp, aggd = _conv_call(hd, hp, srcb, dstb, srcr, dstr)
    hp_new = _combine(aggp, degp, hp, Wm_binds[l], Wr_binds[l], b_binds[l])
    hd_new = _combine(aggd, degd, hd, Wm_rev[l], Wr_rev[l], b_rev[l])
    hd, hp = hd_new, hp_new

  out_d = _proj(hd, Wl_drug, bl_drug, relu=False)
  out_p = _proj(hp, Wl_protein, bl_protein, relu=False)
  return (out_d[:_N], out_p[:_N])
